# Initial kernel scaffold; baseline (speedup 1.0000x reference)
import jax
import jax.numpy as jnp
from jax.experimental import pallas as pl


def kernel(x, edge_index, W1, b1, W2, b2):
    raise NotImplementedError("write your pallas kernel here")


# speedup vs baseline: 10.2531x; 10.2531x over previous
import functools

import jax
import jax.numpy as jnp
from jax import lax
from jax.experimental import pallas as pl
from jax.experimental.pallas import tpu as pltpu
from jax.experimental.pallas import tpu_sc as plsc

N = 10000
E = 320000
D = 128
H = 16
C = 40

NC, NS, L = 2, 16, 16
NW = NC * NS
N_PAD = 10240
PAD_IDX = N
CHUNK = 128
INNER = 8
EPW = 10240
OUTER = EPW // (INNER * CHUNK)
E_PAD = NW * EPW
RPT = N_PAD // NS


def _sc_agg_body(with_deg, *refs):
    if with_deg:
        (table, src, dst, agg_out, deg_out,
         sbuf, dbuf, rows, zbuf, ones_b, accum, deg_accum) = refs
    else:
        (table, src, dst, agg_out,
         sbuf, dbuf, rows, zbuf, accum) = refs
    cid = lax.axis_index("c")
    sid = lax.axis_index("s")
    wid = sid * NC + cid
    row0 = sid * RPT

    def fill(i, _):
        zbuf[i] = jnp.zeros((H,), jnp.float32)
        return 0
    lax.fori_loop(0, RPT, fill, 0)
    pltpu.sync_copy(zbuf, accum.at[pl.ds(row0, RPT)])
    if with_deg:
        pltpu.sync_copy(zbuf, deg_accum.at[pl.ds(row0, RPT)])

        def fill1(i, _):
            ones_b[i] = jnp.ones((H,), jnp.float32)
            return 0
        lax.fori_loop(0, CHUNK, fill1, 0)
    plsc.subcore_barrier()

    nblk = EPW // CHUNK

    def outer(g, _):
        blk = wid * nblk + g * INNER
        pltpu.sync_copy(src.at[pl.ds(blk, INNER)], sbuf)
        pltpu.sync_copy(dst.at[pl.ds(blk, INNER)], dbuf)
        for j in range(INNER):
            pltpu.sync_copy(table.at[sbuf.at[j]], rows)
            pltpu.sync_copy(rows, accum.at[dbuf.at[j]], add=True)
            if with_deg:
                pltpu.sync_copy(ones_b, deg_accum.at[dbuf.at[j]], add=True)
        return 0
    lax.fori_loop(0, OUTER, outer, 0)

    plsc.subcore_barrier()

    out_off = cid * N_PAD + row0
    pltpu.sync_copy(accum.at[pl.ds(row0, RPT)], zbuf)
    pltpu.sync_copy(zbuf, agg_out.at[pl.ds(out_off, RPT)])
    if with_deg:
        pltpu.sync_copy(deg_accum.at[pl.ds(row0, RPT)], zbuf)
        pltpu.sync_copy(zbuf, deg_out.at[pl.ds(out_off, RPT)])


def _make_sc_agg(with_deg):
    mesh = plsc.VectorSubcoreMesh(
        core_axis_name="c", subcore_axis_name="s",
        num_cores=NC, num_subcores=NS)
    outs = [jax.ShapeDtypeStruct((NC * N_PAD, H), jnp.float32)]
    scratch = [
        pltpu.VMEM((INNER, CHUNK), jnp.int32),
        pltpu.VMEM((INNER, CHUNK), jnp.int32),
        pltpu.VMEM((CHUNK, H), jnp.float32),
        pltpu.VMEM((RPT, H), jnp.float32),
    ]
    if with_deg:
        outs.append(jax.ShapeDtypeStruct((NC * N_PAD, H), jnp.float32))
        scratch.append(pltpu.VMEM((CHUNK, H), jnp.float32))
    scratch.append(pltpu.VMEM_SHARED((N_PAD, H), jnp.float32))
    if with_deg:
        scratch.append(pltpu.VMEM_SHARED((N_PAD, H), jnp.float32))
    return pl.kernel(
        functools.partial(_sc_agg_body, with_deg),
        out_type=tuple(outs) if with_deg else outs[0],
        mesh=mesh,
        scratch_types=scratch,
        compiler_params=pltpu.CompilerParams(use_tc_tiling_on_sc=False),
    )


def _mm_body(x_ref, w_ref, o_ref):
    o_ref[...] = jnp.dot(x_ref[...], w_ref[...],
                         preferred_element_type=jnp.float32)


def _h_body(a0, a1, d0, d1, b1_ref, o_ref):
    deg = jnp.maximum(d0[...] + d1[...], 1.0)
    o_ref[...] = jnp.maximum((a0[...] + a1[...]) / deg + b1_ref[...], 0.0)


def _out_body(a0, a1, d0, d1, w_ref, b_ref, o_ref):
    deg = jnp.maximum(d0[...] + d1[...], 1.0)
    mu = (a0[...] + a1[...]) / deg
    z = jnp.dot(mu, w_ref[...], preferred_element_type=jnp.float32) + b_ref[...]
    m = jnp.max(z, axis=1, keepdims=True)
    lse = jnp.log(jnp.sum(jnp.exp(z - m), axis=1, keepdims=True)) + m
    o_ref[...] = z - lse


def kernel(x, edge_index, W1, b1, W2, b2):
    src = edge_index[0]
    dst = edge_index[1]
    pad = jnp.full((E_PAD - E,), PAD_IDX, dtype=jnp.int32)
    src2d = jnp.concatenate([src, pad]).reshape(E_PAD // CHUNK, CHUNK)
    dst2d = jnp.concatenate([dst, pad]).reshape(E_PAD // CHUNK, CHUNK)

    y1 = pl.pallas_call(
        _mm_body,
        grid=(10,),
        in_specs=[pl.BlockSpec((1000, D), lambda i: (i, 0)),
                  pl.BlockSpec((D, H), lambda i: (0, 0))],
        out_specs=pl.BlockSpec((1000, H), lambda i: (i, 0)),
        out_shape=jax.ShapeDtypeStruct((N, H), jnp.float32),
    )(x, W1)
    y1p = jnp.pad(y1, ((0, N_PAD - N), (0, 0)))

    agg1, degp = _make_sc_agg(True)(y1p, src2d, dst2d)

    NB = N_PAD // 1024
    bspec = lambda off: pl.BlockSpec((1024, H), lambda i: (i + off, 0))

    h = pl.pallas_call(
        _h_body,
        grid=(NB,),
        in_specs=[bspec(0), bspec(NB), bspec(0), bspec(NB),
                  pl.BlockSpec((1, H), lambda i: (0, 0))],
        out_specs=pl.BlockSpec((1024, H), lambda i: (i, 0)),
        out_shape=jax.ShapeDtypeStruct((N_PAD, H), jnp.float32),
    )(agg1, agg1, degp, degp, b1.reshape(1, H))

    agg2 = _make_sc_agg(False)(h, src2d, dst2d)

    out = pl.pallas_call(
        _out_body,
        grid=(NB,),
        in_specs=[bspec(0), bspec(NB), bspec(0), bspec(NB),
                  pl.BlockSpec((H, C), lambda i: (0, 0)),
                  pl.BlockSpec((1, C), lambda i: (0, 0))],
        out_specs=pl.BlockSpec((1024, C), lambda i: (i, 0)),
        out_shape=jax.ShapeDtypeStruct((N, C), jnp.float32),
    )(agg2, agg2, degp, degp, W2, b2.reshape(1, C))
    return out


# speedup vs baseline: 14.4888x; 1.4131x over previous
import functools

import jax
import jax.numpy as jnp
from jax import lax
from jax.experimental import pallas as pl
from jax.experimental.pallas import tpu as pltpu
from jax.experimental.pallas import tpu_sc as plsc

N = 10000
E = 320000
D = 128
H = 16
C = 40

NC, NS, L = 2, 16, 16
NW = NC * NS
N_PAD = 10240
PAD_IDX = N
CHUNK = 128
INNER = 8
EPW = 10240
OUTER = EPW // (INNER * CHUNK)
E_PAD = NW * EPW
RPT = N_PAD // NS


def _sc_agg_body(with_deg, *refs):
    if with_deg:
        (table, src, dst, agg_out, deg_out,
         sbuf, dbuf, rows0, rows1, zbuf, ones_b,
         accum, deg_accum,
         sem_i, sem_g0, sem_g1, sem_s0, sem_s1) = refs
    else:
        (table, src, dst, agg_out,
         sbuf, dbuf, rows0, rows1, zbuf,
         accum,
         sem_i, sem_g0, sem_g1, sem_s0, sem_s1) = refs
    rows = (rows0, rows1)
    sem_g, sem_s = (sem_g0, sem_g1), (sem_s0, sem_s1)
    cid = lax.axis_index("c")
    sid = lax.axis_index("s")
    wid = sid * NC + cid
    row0 = sid * RPT
    nblk = EPW // CHUNK
    blk0 = wid * nblk

    pltpu.async_copy(src.at[pl.ds(blk0, nblk)], sbuf, sem_i)
    pltpu.async_copy(dst.at[pl.ds(blk0, nblk)], dbuf, sem_i)

    def fill(i, _):
        zbuf[i] = jnp.zeros((H,), jnp.float32)
        return 0
    lax.fori_loop(0, RPT, fill, 0)
    pltpu.sync_copy(zbuf, accum.at[pl.ds(row0, RPT)])
    if with_deg:
        pltpu.sync_copy(zbuf, deg_accum.at[pl.ds(row0, RPT)])

        def fill1(i, _):
            ones_b[i] = jnp.ones((H,), jnp.float32)
            return 0
        lax.fori_loop(0, CHUNK, fill1, 0)
    pltpu.make_async_copy(src.at[pl.ds(blk0, nblk)], sbuf, sem_i).wait()
    pltpu.make_async_copy(dst.at[pl.ds(blk0, nblk)], dbuf, sem_i).wait()
    plsc.subcore_barrier()

    def drain_scatters(p):
        for j in range(INNER):
            pltpu.make_async_copy(
                rows[p].at[j], accum.at[dbuf.at[j]], sem_s[p]).wait()
            if with_deg:
                pltpu.make_async_copy(
                    ones_b, deg_accum.at[dbuf.at[j]], sem_s[p]).wait()

    def outer(i, _):
        for p in range(2):
            g = 2 * i + p

            @pl.when(i > 0)
            def _():
                drain_scatters(p)
            gd = [pltpu.async_copy(table.at[sbuf.at[g * INNER + j]],
                                   rows[p].at[j], sem_g[p])
                  for j in range(INNER)]
            for j in range(INNER):
                gd[j].wait()
            for j in range(INNER):
                pltpu.async_copy(rows[p].at[j],
                                 accum.at[dbuf.at[g * INNER + j]],
                                 sem_s[p], add=True)
                if with_deg:
                    pltpu.async_copy(ones_b,
                                     deg_accum.at[dbuf.at[g * INNER + j]],
                                     sem_s[p], add=True)
        return 0
    lax.fori_loop(0, OUTER // 2, outer, 0)
    for p in range(2):
        drain_scatters(p)

    plsc.subcore_barrier()

    out_off = cid * N_PAD + row0
    pltpu.sync_copy(accum.at[pl.ds(row0, RPT)], zbuf)
    pltpu.sync_copy(zbuf, agg_out.at[pl.ds(out_off, RPT)])
    if with_deg:
        pltpu.sync_copy(deg_accum.at[pl.ds(row0, RPT)], zbuf)
        pltpu.sync_copy(zbuf, deg_out.at[pl.ds(out_off, RPT)])


def _make_sc_agg(with_deg):
    mesh = plsc.VectorSubcoreMesh(
        core_axis_name="c", subcore_axis_name="s",
        num_cores=NC, num_subcores=NS)
    outs = [jax.ShapeDtypeStruct((NC * N_PAD, H), jnp.float32)]
    scratch = [
        pltpu.VMEM((EPW // CHUNK, CHUNK), jnp.int32),
        pltpu.VMEM((EPW // CHUNK, CHUNK), jnp.int32),
        pltpu.VMEM((INNER, CHUNK, H), jnp.float32),
        pltpu.VMEM((INNER, CHUNK, H), jnp.float32),
        pltpu.VMEM((RPT, H), jnp.float32),
    ]
    if with_deg:
        outs.append(jax.ShapeDtypeStruct((NC * N_PAD, H), jnp.float32))
        scratch.append(pltpu.VMEM((CHUNK, H), jnp.float32))
    scratch.append(pltpu.VMEM_SHARED((N_PAD, H), jnp.float32))
    if with_deg:
        scratch.append(pltpu.VMEM_SHARED((N_PAD, H), jnp.float32))
    scratch.extend([pltpu.SemaphoreType.DMA] * 5)
    return pl.kernel(
        functools.partial(_sc_agg_body, with_deg),
        out_type=tuple(outs) if with_deg else outs[0],
        mesh=mesh,
        scratch_types=scratch,
        compiler_params=pltpu.CompilerParams(use_tc_tiling_on_sc=False),
    )


def _mm_body(x_ref, w_ref, o_ref):
    o_ref[...] = jnp.dot(x_ref[...], w_ref[...],
                         preferred_element_type=jnp.float32)


def _h_body(a0, a1, d0, d1, b1_ref, o_ref):
    deg = jnp.maximum(d0[...] + d1[...], 1.0)
    o_ref[...] = jnp.maximum((a0[...] + a1[...]) / deg + b1_ref[...], 0.0)


def _out_body(a0, a1, d0, d1, w_ref, b_ref, o_ref):
    deg = jnp.maximum(d0[...] + d1[...], 1.0)
    mu = (a0[...] + a1[...]) / deg
    z = jnp.dot(mu, w_ref[...], preferred_element_type=jnp.float32) + b_ref[...]
    m = jnp.max(z, axis=1, keepdims=True)
    lse = jnp.log(jnp.sum(jnp.exp(z - m), axis=1, keepdims=True)) + m
    o_ref[...] = z - lse


def kernel(x, edge_index, W1, b1, W2, b2):
    src = edge_index[0]
    dst = edge_index[1]
    NB1 = N_PAD // 1024
    pad = jnp.full((E_PAD - E,), PAD_IDX, dtype=jnp.int32)
    src2d = jnp.concatenate([src, pad]).reshape(E_PAD // CHUNK, CHUNK)
    dst2d = jnp.concatenate([dst, pad]).reshape(E_PAD // CHUNK, CHUNK)

    y1p = pl.pallas_call(
        _mm_body,
        grid=(NB1,),
        in_specs=[pl.BlockSpec((1024, D), lambda i: (i, 0)),
                  pl.BlockSpec((D, H), lambda i: (0, 0))],
        out_specs=pl.BlockSpec((1024, H), lambda i: (i, 0)),
        out_shape=jax.ShapeDtypeStruct((N_PAD, H), jnp.float32),
    )(x, W1)

    agg1, degp = _make_sc_agg(True)(y1p, src2d, dst2d)

    NB = N_PAD // 1024
    bspec = lambda off: pl.BlockSpec((1024, H), lambda i: (i + off, 0))

    h = pl.pallas_call(
        _h_body,
        grid=(NB,),
        in_specs=[bspec(0), bspec(NB), bspec(0), bspec(NB),
                  pl.BlockSpec((1, H), lambda i: (0, 0))],
        out_specs=pl.BlockSpec((1024, H), lambda i: (i, 0)),
        out_shape=jax.ShapeDtypeStruct((N_PAD, H), jnp.float32),
    )(agg1, agg1, degp, degp, b1.reshape(1, H))

    agg2 = _make_sc_agg(False)(h, src2d, dst2d)

    out = pl.pallas_call(
        _out_body,
        grid=(NB,),
        in_specs=[bspec(0), bspec(NB), bspec(0), bspec(NB),
                  pl.BlockSpec((H, C), lambda i: (0, 0)),
                  pl.BlockSpec((1, C), lambda i: (0, 0))],
        out_specs=pl.BlockSpec((1024, C), lambda i: (i, 0)),
        out_shape=jax.ShapeDtypeStruct((N, C), jnp.float32),
    )(agg2, agg2, degp, degp, W2, b2.reshape(1, C))
    return out


# speedup vs baseline: 23.6947x; 1.6354x over previous
import functools

import jax
import jax.numpy as jnp
from jax import lax
from jax.experimental import pallas as pl
from jax.experimental.pallas import tpu as pltpu
from jax.experimental.pallas import tpu_sc as plsc

N = 10000
E = 320000
D = 128
H = 16
C = 40

NC, NS, L = 2, 16, 16
NW = NC * NS
N_PAD = 10240
PAD_IDX = N
CHUNK = 128
INNER = 8
EPW = 10240
OUTER = EPW // (INNER * CHUNK)
E_PAD = NW * EPW
RPT = N_PAD // NS


def _sc_agg_body(with_deg, *refs):
    if with_deg:
        (table, src, dst, agg_out, deg_out,
         sbuf, dbuf, rows0, rows1, zbuf, ones_b,
         accum, deg_accum,
         sem_i, sem_g0, sem_g1, sem_s0, sem_s1) = refs
    else:
        (table, src, dst, agg_out,
         sbuf, dbuf, rows0, rows1, zbuf,
         accum,
         sem_i, sem_g0, sem_g1, sem_s0, sem_s1) = refs
    rows = (rows0, rows1)
    sem_g, sem_s = (sem_g0, sem_g1), (sem_s0, sem_s1)
    cid = lax.axis_index("c")
    sid = lax.axis_index("s")
    wid = sid * NC + cid
    row0 = sid * RPT
    nblk = EPW // CHUNK
    blk0 = wid * nblk

    pltpu.async_copy(src.at[pl.ds(blk0, nblk)], sbuf, sem_i)
    pltpu.async_copy(dst.at[pl.ds(blk0, nblk)], dbuf, sem_i)

    def fill(i, _):
        zbuf[i] = jnp.zeros((H,), jnp.float32)
        return 0
    lax.fori_loop(0, RPT, fill, 0)
    pltpu.sync_copy(zbuf, accum.at[pl.ds(row0, RPT)])
    if with_deg:
        pltpu.sync_copy(zbuf, deg_accum.at[pl.ds(row0, RPT)])

        def fill1(i, _):
            ones_b[i] = jnp.ones((H,), jnp.float32)
            return 0
        lax.fori_loop(0, CHUNK, fill1, 0)
    pltpu.make_async_copy(src.at[pl.ds(blk0, nblk)], sbuf, sem_i).wait()
    pltpu.make_async_copy(dst.at[pl.ds(blk0, nblk)], dbuf, sem_i).wait()
    plsc.subcore_barrier()

    def drain_scatters(p):
        for j in range(INNER):
            pltpu.make_async_copy(
                rows[p].at[j], accum.at[dbuf.at[j]], sem_s[p]).wait()
            if with_deg:
                pltpu.make_async_copy(
                    ones_b, deg_accum.at[dbuf.at[j]], sem_s[p]).wait()

    def outer(i, _):
        for p in range(2):
            g = 2 * i + p

            @pl.when(i > 0)
            def _():
                drain_scatters(p)
            gd = [pltpu.async_copy(table.at[sbuf.at[g * INNER + j]],
                                   rows[p].at[j], sem_g[p])
                  for j in range(INNER)]
            for j in range(INNER):
                gd[j].wait()
            for j in range(INNER):
                pltpu.async_copy(rows[p].at[j],
                                 accum.at[dbuf.at[g * INNER + j]],
                                 sem_s[p], add=True)
                if with_deg:
                    pltpu.async_copy(ones_b,
                                     deg_accum.at[dbuf.at[g * INNER + j]],
                                     sem_s[p], add=True)
        return 0
    lax.fori_loop(0, OUTER // 2, outer, 0)
    for p in range(2):
        drain_scatters(p)

    plsc.subcore_barrier()

    out_off = cid * N_PAD + row0
    pltpu.sync_copy(accum.at[pl.ds(row0, RPT)], zbuf)
    pltpu.sync_copy(zbuf, agg_out.at[pl.ds(out_off, RPT)])
    if with_deg:
        pltpu.sync_copy(deg_accum.at[pl.ds(row0, RPT)], zbuf)
        pltpu.sync_copy(zbuf, deg_out.at[pl.ds(out_off, RPT)])


def _make_sc_agg(with_deg):
    mesh = plsc.VectorSubcoreMesh(
        core_axis_name="c", subcore_axis_name="s",
        num_cores=NC, num_subcores=NS)
    outs = [jax.ShapeDtypeStruct((NC * N_PAD, H), jnp.float32)]
    scratch = [
        pltpu.VMEM((EPW // CHUNK, CHUNK), jnp.int32),
        pltpu.VMEM((EPW // CHUNK, CHUNK), jnp.int32),
        pltpu.VMEM((INNER, CHUNK, H), jnp.float32),
        pltpu.VMEM((INNER, CHUNK, H), jnp.float32),
        pltpu.VMEM((RPT, H), jnp.float32),
    ]
    if with_deg:
        outs.append(jax.ShapeDtypeStruct((NC * N_PAD, H), jnp.float32))
        scratch.append(pltpu.VMEM((CHUNK, H), jnp.float32))
    scratch.append(pltpu.VMEM_SHARED((N_PAD, H), jnp.float32))
    if with_deg:
        scratch.append(pltpu.VMEM_SHARED((N_PAD, H), jnp.float32))
    scratch.extend([pltpu.SemaphoreType.DMA] * 5)
    return pl.kernel(
        functools.partial(_sc_agg_body, with_deg),
        out_type=tuple(outs) if with_deg else outs[0],
        mesh=mesh,
        scratch_types=scratch,
        compiler_params=pltpu.CompilerParams(use_tc_tiling_on_sc=False),
    )


def _mm_body(x_ref, w_ref, o_ref):
    o_ref[...] = jnp.dot(x_ref[...], w_ref[...],
                         preferred_element_type=jnp.float32)


def _h_body(a0, a1, d0, d1, b1_ref, o_ref):
    deg = jnp.maximum(d0[...] + d1[...], 1.0)
    o_ref[...] = jnp.maximum((a0[...] + a1[...]) / deg + b1_ref[...], 0.0)


def _out_body(a0, a1, d0, d1, w_ref, b_ref, o_ref):
    deg = jnp.maximum(d0[...] + d1[...], 1.0)
    mu = (a0[...] + a1[...]) / deg
    z = jnp.dot(mu, w_ref[...], preferred_element_type=jnp.float32) + b_ref[...]
    m = jnp.max(z, axis=1, keepdims=True)
    lse = jnp.log(jnp.sum(jnp.exp(z - m), axis=1, keepdims=True)) + m
    o_ref[...] = z - lse


def kernel(x, edge_index, W1, b1, W2, b2):
    src = edge_index[0]
    dst = edge_index[1]
    NB1 = N_PAD // 1024
    pad = PAD_IDX + (jnp.arange(E_PAD - E, dtype=jnp.int32) % (N_PAD - N))
    src2d = jnp.concatenate([src, pad]).reshape(E_PAD // CHUNK, CHUNK)
    dst2d = jnp.concatenate([dst, pad]).reshape(E_PAD // CHUNK, CHUNK)

    y1p = pl.pallas_call(
        _mm_body,
        grid=(NB1,),
        in_specs=[pl.BlockSpec((1024, D), lambda i: (i, 0)),
                  pl.BlockSpec((D, H), lambda i: (0, 0))],
        out_specs=pl.BlockSpec((1024, H), lambda i: (i, 0)),
        out_shape=jax.ShapeDtypeStruct((N_PAD, H), jnp.float32),
    )(x, W1)

    agg1, degp = _make_sc_agg(True)(y1p, src2d, dst2d)

    NB = N_PAD // 1024
    bspec = lambda off: pl.BlockSpec((1024, H), lambda i: (i + off, 0))

    h = pl.pallas_call(
        _h_body,
        grid=(NB,),
        in_specs=[bspec(0), bspec(NB), bspec(0), bspec(NB),
                  pl.BlockSpec((1, H), lambda i: (0, 0))],
        out_specs=pl.BlockSpec((1024, H), lambda i: (i, 0)),
        out_shape=jax.ShapeDtypeStruct((N_PAD, H), jnp.float32),
    )(agg1, agg1, degp, degp, b1.reshape(1, H))

    agg2 = _make_sc_agg(False)(h, src2d, dst2d)

    out = pl.pallas_call(
        _out_body,
        grid=(NB,),
        in_specs=[bspec(0), bspec(NB), bspec(0), bspec(NB),
                  pl.BlockSpec((H, C), lambda i: (0, 0)),
                  pl.BlockSpec((1, C), lambda i: (0, 0))],
        out_specs=pl.BlockSpec((1024, C), lambda i: (i, 0)),
        out_shape=jax.ShapeDtypeStruct((N, C), jnp.float32),
    )(agg2, agg2, degp, degp, W2, b2.reshape(1, C))
    return out


# speedup vs baseline: 30.8130x; 1.3004x over previous
import functools

import jax
import jax.numpy as jnp
from jax import lax
from jax.experimental import pallas as pl
from jax.experimental.pallas import tpu as pltpu
from jax.experimental.pallas import tpu_sc as plsc

N = 10000
E = 320000
D = 128
H = 16
C = 40

NC, NS, L = 2, 16, 16
NW = NC * NS
N_PAD = 10240
CHUNK = 128
NCH = E // CHUNK
CH_BASE = NCH // NW
CH_EXTRA = NCH - CH_BASE * NW
RPT = N_PAD // NS
NSLOT = 4
NSTEP = 4 * ((CH_BASE + 1 + 2) // 4 + 1)


def _worker_range(wid):
    nch = CH_BASE + (wid < CH_EXTRA).astype(jnp.int32)
    ch0 = wid * CH_BASE + jnp.minimum(wid, CH_EXTRA)
    return ch0, nch


def _load_idx(e2d, sbuf, dbuf, sem_i, ch0, wid):
    pltpu.async_copy(e2d.at[0, pl.ds(ch0, CH_BASE)],
                     sbuf.at[pl.ds(0, CH_BASE)], sem_i)
    pltpu.async_copy(e2d.at[1, pl.ds(ch0, CH_BASE)],
                     dbuf.at[pl.ds(0, CH_BASE)], sem_i)

    @pl.when(wid < CH_EXTRA)
    def _():
        pltpu.async_copy(e2d.at[0, pl.ds(ch0 + CH_BASE, 1)],
                         sbuf.at[pl.ds(CH_BASE, 1)], sem_i)
        pltpu.async_copy(e2d.at[1, pl.ds(ch0 + CH_BASE, 1)],
                         dbuf.at[pl.ds(CH_BASE, 1)], sem_i)


def _drain_idx(e2d, sbuf, dbuf, sem_i, ch0, wid):
    pltpu.make_async_copy(e2d.at[0, pl.ds(ch0, CH_BASE)],
                          sbuf.at[pl.ds(0, CH_BASE)], sem_i).wait()
    pltpu.make_async_copy(e2d.at[1, pl.ds(ch0, CH_BASE)],
                          dbuf.at[pl.ds(0, CH_BASE)], sem_i).wait()

    @pl.when(wid < CH_EXTRA)
    def _():
        pltpu.make_async_copy(e2d.at[0, pl.ds(ch0 + CH_BASE, 1)],
                              sbuf.at[pl.ds(CH_BASE, 1)], sem_i).wait()
        pltpu.make_async_copy(e2d.at[1, pl.ds(ch0 + CH_BASE, 1)],
                              dbuf.at[pl.ds(CH_BASE, 1)], sem_i).wait()


def _agg_pipeline(table, sbuf, dbuf, rows, nch, accum, deg_accum, ones_b,
                  sem_g, sem_s):
    def step(c, q):
        q2 = (q + 2) % NSLOT

        @pl.when(jnp.logical_and(c >= 2, c - 2 < nch))
        def _():
            pltpu.make_async_copy(
                rows.at[q2], accum.at[dbuf.at[0]], sem_s[q2]).wait()
            if deg_accum is not None:
                pltpu.make_async_copy(
                    ones_b, deg_accum.at[dbuf.at[0]], sem_s[q2]).wait()

        @pl.when(c + 2 < nch)
        def _():
            pltpu.async_copy(table.at[sbuf.at[c + 2]], rows.at[q2],
                             sem_g[q2])

        @pl.when(c < nch)
        def _():
            pltpu.make_async_copy(
                table.at[sbuf.at[0]], rows.at[q], sem_g[q]).wait()
            pltpu.async_copy(rows.at[q], accum.at[dbuf.at[c]],
                             sem_s[q], add=True)
            if deg_accum is not None:
                pltpu.async_copy(ones_b, deg_accum.at[dbuf.at[c]],
                                 sem_s[q], add=True)

    pltpu.async_copy(table.at[sbuf.at[0]], rows.at[0], sem_g[0])
    pltpu.async_copy(table.at[sbuf.at[1]], rows.at[1], sem_g[1])

    def outer(i, _):
        for q in range(NSLOT):
            step(i * NSLOT + q, q)
        return 0
    lax.fori_loop(0, NSTEP // NSLOT, outer, 0)


def _zero_fill(buf, n):
    def f(i, _):
        buf[i] = jnp.zeros((H,), jnp.float32)
        return 0
    lax.fori_loop(0, n, f, 0)


def _sc_l1_body(table, e2d, agg_out, deg_out,
                sbuf, dbuf, rows, zbuf, ones_b, accum, deg_accum,
                sem_i, sg0, sg1, sg2, sg3, ss0, ss1, ss2, ss3):
    sem_g, sem_s = (sg0, sg1, sg2, sg3), (ss0, ss1, ss2, ss3)
    cid = lax.axis_index("c")
    sid = lax.axis_index("s")
    wid = sid * NC + cid
    row0 = sid * RPT
    ch0, nch = _worker_range(wid)

    _load_idx(e2d, sbuf, dbuf, sem_i, ch0, wid)
    _zero_fill(zbuf, RPT)
    pltpu.sync_copy(zbuf, accum.at[pl.ds(row0, RPT)])
    pltpu.sync_copy(zbuf, deg_accum.at[pl.ds(row0, RPT)])

    def f1(i, _):
        ones_b[i] = jnp.ones((H,), jnp.float32)
        return 0
    lax.fori_loop(0, CHUNK, f1, 0)
    _drain_idx(e2d, sbuf, dbuf, sem_i, ch0, wid)
    plsc.subcore_barrier()

    _agg_pipeline(table, sbuf, dbuf, rows, nch, accum, deg_accum, ones_b,
                  sem_g, sem_s)

    plsc.subcore_barrier()
    out_off = cid * N_PAD + row0
    pltpu.sync_copy(accum.at[pl.ds(row0, RPT)], zbuf)
    pltpu.sync_copy(zbuf, agg_out.at[pl.ds(out_off, RPT)])
    pltpu.sync_copy(deg_accum.at[pl.ds(row0, RPT)], zbuf)
    pltpu.sync_copy(zbuf, deg_out.at[pl.ds(out_off, RPT)])


def _sc_l2_body(agg_in, deg_in, e2d, b1h, mu_out,
                sbuf, dbuf, rows, zbuf, a0, a1, d0, d1, b1v, htab, accum,
                sem_i, sem_p, sg0, sg1, sg2, sg3, ss0, ss1, ss2, ss3):
    sem_g, sem_s = (sg0, sg1, sg2, sg3), (ss0, ss1, ss2, ss3)
    cid = lax.axis_index("c")
    sid = lax.axis_index("s")
    wid = sid * NC + cid
    row0 = sid * RPT
    ch0, nch = _worker_range(wid)

    _load_idx(e2d, sbuf, dbuf, sem_i, ch0, wid)
    pltpu.async_copy(agg_in.at[pl.ds(row0, RPT)], a0, sem_p)
    pltpu.async_copy(agg_in.at[pl.ds(N_PAD + row0, RPT)], a1, sem_p)
    pltpu.async_copy(deg_in.at[pl.ds(row0, RPT)], d0, sem_p)
    pltpu.async_copy(deg_in.at[pl.ds(N_PAD + row0, RPT)], d1, sem_p)
    pltpu.async_copy(b1h, b1v, sem_p)

    _zero_fill(zbuf, RPT)
    pltpu.sync_copy(zbuf, accum.at[pl.ds(row0, RPT)])

    pltpu.make_async_copy(agg_in.at[pl.ds(row0, RPT)], a0, sem_p).wait()
    pltpu.make_async_copy(agg_in.at[pl.ds(row0, RPT)], a1, sem_p).wait()
    pltpu.make_async_copy(deg_in.at[pl.ds(row0, RPT)], d0, sem_p).wait()
    pltpu.make_async_copy(deg_in.at[pl.ds(row0, RPT)], d1, sem_p).wait()
    pltpu.make_async_copy(b1h, b1v, sem_p).wait()

    bvec = b1v[0]

    def hrow(i, _):
        deg = jnp.maximum(d0[i] + d1[i], 1.0)
        a0[i] = jnp.maximum((a0[i] + a1[i]) / deg + bvec, 0.0)
        return 0
    lax.fori_loop(0, RPT, hrow, 0)
    pltpu.sync_copy(a0, htab.at[pl.ds(row0, RPT)])
    _drain_idx(e2d, sbuf, dbuf, sem_i, ch0, wid)
    plsc.subcore_barrier()

    _agg_pipeline(htab, sbuf, dbuf, rows, nch, accum, None, None,
                  sem_g, sem_s)

    plsc.subcore_barrier()
    pltpu.sync_copy(accum.at[pl.ds(row0, RPT)], zbuf)

    def mrow(i, _):
        deg = jnp.maximum(d0[i] + d1[i], 1.0)
        zbuf[i] = zbuf[i] / deg
        return 0
    lax.fori_loop(0, RPT, mrow, 0)
    pltpu.sync_copy(zbuf, mu_out.at[pl.ds(cid * N_PAD + row0, RPT)])


_SC_MESH = plsc.VectorSubcoreMesh(
    core_axis_name="c", subcore_axis_name="s",
    num_cores=NC, num_subcores=NS)
_SC_PARAMS = pltpu.CompilerParams(use_tc_tiling_on_sc=False)


def _make_sc_l1():
    scratch = [
        pltpu.VMEM((CH_BASE + 1, CHUNK), jnp.int32),
        pltpu.VMEM((CH_BASE + 1, CHUNK), jnp.int32),
        pltpu.VMEM((NSLOT, CHUNK, H), jnp.float32),
        pltpu.VMEM((RPT, H), jnp.float32),
        pltpu.VMEM((CHUNK, H), jnp.float32),
        pltpu.VMEM_SHARED((N_PAD, H), jnp.float32),
        pltpu.VMEM_SHARED((N_PAD, H), jnp.float32),
    ] + [pltpu.SemaphoreType.DMA] * 9
    return pl.kernel(
        _sc_l1_body,
        out_type=(jax.ShapeDtypeStruct((NC * N_PAD, H), jnp.float32),
                  jax.ShapeDtypeStruct((NC * N_PAD, H), jnp.float32)),
        mesh=_SC_MESH,
        scratch_types=scratch,
        compiler_params=_SC_PARAMS,
    )


def _make_sc_l2():
    scratch = [
        pltpu.VMEM((CH_BASE + 1, CHUNK), jnp.int32),
        pltpu.VMEM((CH_BASE + 1, CHUNK), jnp.int32),
        pltpu.VMEM((NSLOT, CHUNK, H), jnp.float32),
        pltpu.VMEM((RPT, H), jnp.float32),
        pltpu.VMEM((RPT, H), jnp.float32),
        pltpu.VMEM((RPT, H), jnp.float32),
        pltpu.VMEM((RPT, H), jnp.float32),
        pltpu.VMEM((RPT, H), jnp.float32),
        pltpu.VMEM((1, H), jnp.float32),
        pltpu.VMEM_SHARED((N_PAD, H), jnp.float32),
        pltpu.VMEM_SHARED((N_PAD, H), jnp.float32),
    ] + [pltpu.SemaphoreType.DMA] * 10
    return pl.kernel(
        _sc_l2_body,
        out_type=jax.ShapeDtypeStruct((NC * N_PAD, H), jnp.float32),
        mesh=_SC_MESH,
        scratch_types=scratch,
        compiler_params=_SC_PARAMS,
    )


def _mm_body(x_ref, w_ref, o_ref):
    o_ref[...] = jnp.dot(x_ref[...], w_ref[...],
                         preferred_element_type=jnp.float32)


def _out_body(m0, m1, w_ref, b_ref, o_ref):
    z = jnp.dot(m0[...] + m1[...], w_ref[...],
                preferred_element_type=jnp.float32) + b_ref[...]
    m = jnp.max(z, axis=1, keepdims=True)
    lse = jnp.log(jnp.sum(jnp.exp(z - m), axis=1, keepdims=True)) + m
    o_ref[...] = z - lse


def kernel(x, edge_index, W1, b1, W2, b2):
    e2d = edge_index.reshape(2, NCH, CHUNK)
    NB = N_PAD // 1024

    y1p = pl.pallas_call(
        _mm_body,
        grid=(NB,),
        in_specs=[pl.BlockSpec((1024, D), lambda i: (i, 0)),
                  pl.BlockSpec((D, H), lambda i: (0, 0))],
        out_specs=pl.BlockSpec((1024, H), lambda i: (i, 0)),
        out_shape=jax.ShapeDtypeStruct((N_PAD, H), jnp.float32),
    )(x, W1)

    agg1, degp = _make_sc_l1()(y1p, e2d)

    mu2 = _make_sc_l2()(agg1, degp, e2d, b1.reshape(1, H))

    bspec = lambda off: pl.BlockSpec((1024, H), lambda i: (i + off, 0))
    out = pl.pallas_call(
        _out_body,
        grid=(NB,),
        in_specs=[bspec(0), bspec(NB),
                  pl.BlockSpec((H, C), lambda i: (0, 0)),
                  pl.BlockSpec((1, C), lambda i: (0, 0))],
        out_specs=pl.BlockSpec((1024, C), lambda i: (i, 0)),
        out_shape=jax.ShapeDtypeStruct((N, C), jnp.float32),
    )(mu2, mu2, W2, b2.reshape(1, C))
    return out
